# baseline (speedup 1.0000x reference)
import functools

import jax
import jax.numpy as jnp
from jax import lax
from jax.experimental import pallas as pl
from jax.experimental.pallas import tpu as pltpu
from jax.experimental.pallas import tpu_sc as plsc

B = 16384
C = 1000
NBINS = 10
MINCLAMP = 0.01
MAXCLAMP = 100.0

NC = 2
NS = 16
L = 16
NW = NC * NS
RW = B // NW

EDGES = (0.0, 0.10000000149011612, 0.20000000298023224, 0.30000001192092896,
         0.4000000059604645, 0.5, 0.6000000238418579, 0.699999988079071,
         0.800000011920929, 0.9000000357627869, 1.0)

CBLK = 1024


def _rowmax_tc_body(p_ref, out_ref, mm_ref, z_ref, accn_ref, accx_ref):
    g = pl.program_id(0)
    x = p_ref[...]
    m = jnp.max(x, axis=0).reshape(CBLK // 128, 128)
    out_ref[...] = m
    z_ref[...] = jnp.zeros((C, CBLK), jnp.float32)

    @pl.when(g == 0)
    def _():
        accn_ref[...] = m
        accx_ref[...] = m

    @pl.when(g > 0)
    def _():
        accn_ref[...] = jnp.minimum(accn_ref[...], m)
        accx_ref[...] = jnp.maximum(accx_ref[...], m)

    @pl.when(g == pl.num_programs(0) - 1)
    def _():
        gmn = jnp.min(accn_ref[...])
        gmx = jnp.max(accx_ref[...])
        col = lax.broadcasted_iota(jnp.int32, (8, 128), 1)
        mm_ref[...] = jnp.where(col == 1, gmx, gmn)


def _rowmax_tc(probs_t):
    return pl.pallas_call(
        _rowmax_tc_body,
        grid=(B // CBLK,),
        in_specs=[pl.BlockSpec((C, CBLK), lambda g: (0, g))],
        out_specs=[
            pl.BlockSpec((CBLK // 128, 128), lambda g: (g, 0)),
            pl.BlockSpec((8, 128), lambda g: (0, 0)),
            pl.BlockSpec((C, CBLK), lambda g: (0, g)),
        ],
        out_shape=[
            jax.ShapeDtypeStruct((B // 128, 128), jnp.float32),
            jax.ShapeDtypeStruct((8, 128), jnp.float32),
            jax.ShapeDtypeStruct((C, B), jnp.float32),
        ],
        scratch_shapes=[
            pltpu.VMEM((CBLK // 128, 128), jnp.float32),
            pltpu.VMEM((CBLK // 128, 128), jnp.float32),
        ],
    )(probs_t)


@functools.partial(
    pl.kernel,
    out_type=jax.ShapeDtypeStruct((B,), jnp.float32),
    mesh=plsc.VectorSubcoreMesh(core_axis_name="c", subcore_axis_name="s",
                                num_cores=NC, num_subcores=NS),
    scratch_types=[
        pltpu.VMEM((RW,), jnp.float32),
        pltpu.VMEM((L,), jnp.float32),
        pltpu.VMEM((L,), jnp.float32),
        pltpu.VMEM((L,), jnp.float32),
        pltpu.VMEM((RW,), jnp.float32),
    ],
    compiler_params=pltpu.CompilerParams(needs_layout_passes=False),
)
def _calibrate_sc(maxes_hbm, mm_hbm, bp_hbm, out_hbm,
                  m_v, mm_v, bp_v, sc_v, out_v):
    wid = lax.axis_index("c") * NS + lax.axis_index("s")
    rbase = wid * RW

    pltpu.sync_copy(maxes_hbm.at[pl.ds(rbase, RW)], m_v)
    pltpu.sync_copy(mm_hbm.at[0, pl.ds(0, L)], mm_v)
    pltpu.sync_copy(bp_hbm, bp_v.at[pl.ds(0, NBINS)])

    mm = mm_v[:]
    gmn = mm[0]
    gmx = mm[1]
    denom_v = jnp.zeros((L,), jnp.float32) + (gmx - gmn + jnp.float32(1e-12))
    inv = jnp.full((L,), 1.0, jnp.float32) / denom_v

    bp = bp_v[:]
    sc_v[:] = jnp.float32(1.0) / jnp.clip(bp * bp, jnp.float32(MINCLAMP),
                                          jnp.float32(MAXCLAMP))

    def vec_body(k, carry):
        x = (m_v[pl.ds(k * L, L)] - gmn) * inv
        cnt = jnp.zeros((L,), jnp.int32)
        for e in EDGES:
            cnt = cnt + jnp.where(jnp.float32(e) < x, jnp.int32(1),
                                  jnp.int32(0))
        idx = jnp.clip(cnt - 1, 0, NBINS - 1)
        g = plsc.load_gather(sc_v, [idx])
        out_v[pl.ds(k * L, L)] = x * g
        return carry

    lax.fori_loop(0, RW // L, vec_body, 0)
    pltpu.sync_copy(out_v, out_hbm.at[pl.ds(rbase, RW)])


def kernel(probabilities, bin_params):
    maxes2d, mm2d, zeros_t = _rowmax_tc(probabilities.T)
    out = _calibrate_sc(maxes2d.reshape(B), mm2d, bin_params)
    calibrated = zeros_t.T
    return (out, calibrated)
